# baseline (speedup 1.0000x reference)
import functools

import jax
import jax.numpy as jnp
from jax import lax
from jax.experimental import pallas as pl
from jax.experimental.pallas import tpu as pltpu
from jax.experimental.pallas import tpu_sc as plsc

EMBED = 64
NB = 4096
NT = 200
NUM_WORKERS = 32
BBLK = 256
BLOCKS_PER_T = NB // BBLK
N_BLOCKS = NT * BLOCKS_PER_T
BLOCKS_PER_W = N_BLOCKS // NUM_WORKERS
NIDX = 4
NROW = 2
BBLK_P = BBLK + 1
GROUP = 4


def _gather_body(ev_hbm, table_hbm, out_hbm, idxv, rows, rows_t, isem, gsem,
                 wsem):
    wid = lax.axis_index("s") * 2 + lax.axis_index("c")
    k0 = wid * BLOCKS_PER_W

    iota16 = lax.iota(jnp.int32, 16)
    evecs = [q * 16 + iota16 for q in range(4)]

    def t_of(i):
        return (k0 + i) // BLOCKS_PER_T

    def b0_of(i):
        return ((k0 + i) % BLOCKS_PER_T) * BBLK

    def start_idx(i, s):
        pltpu.make_async_copy(
            ev_hbm.at[t_of(i), pl.ds(b0_of(i), BBLK)], idxv.at[s], isem.at[s]
        ).start()

    def wait_idx(s):
        pltpu.make_async_copy(
            ev_hbm.at[0, pl.ds(0, BBLK)], idxv.at[s], isem.at[s]
        ).wait()

    def start_gather(s, b):
        pltpu.make_async_copy(
            table_hbm.at[idxv.at[s]], rows.at[b], gsem.at[b]
        ).start()

    def wait_gather(b):
        pltpu.make_async_copy(
            table_hbm.at[idxv.at[0]], rows.at[b], gsem.at[b]
        ).wait()

    def start_write(i):
        t = t_of(i)
        c0 = b0_of(i) // 128

        def one(e, _):
            pltpu.make_async_copy(
                rows_t.at[:, pl.ds(e, 1), pl.ds(0, 128)],
                out_hbm.at[t, e // 8, pl.ds(c0, BBLK // 128),
                           pl.ds(e % 8, 1), :],
                wsem
            ).start()
            return ()

        lax.fori_loop(0, EMBED, one, ())

    def wait_write():
        def one(e, _):
            pltpu.make_async_copy(
                rows_t.at[:, pl.ds(e, 1), pl.ds(0, 128)],
                out_hbm.at[0, 0, pl.ds(0, BBLK // 128), pl.ds(0, 1), :], wsem
            ).wait()
            return ()

        lax.fori_loop(0, EMBED, one, ())

    def transpose(b):
        rows_b = rows.at[b]

        @plsc.parallel_loop(0, BBLK, step=1, unroll=4)
        def _(j):
            cv = jnp.full((16,), j // 128, jnp.int32)
            bv = jnp.full((16,), j % 128, jnp.int32)
            for q in range(4):
                val = rows_b[j, pl.ds(q * 16, 16)]
                plsc.store_scatter(rows_t, [cv, evecs[q], bv], val)

    def step(i, ib, *, first, next_gather, next_idx):
        if next_gather:
            wait_idx((ib + 1) % NIDX)
            start_gather((ib + 1) % NIDX, (ib + 1) % NROW)
        if next_idx:
            start_idx(i + 3, (ib + 3) % NIDX)
        wait_gather(ib % NROW)
        if not first:
            wait_write()
        transpose(ib % NROW)
        start_write(i)

    for s in range(3):
        start_idx(s, s)
    wait_idx(0)
    start_gather(0, 0)

    step(0, 0, first=True, next_gather=True, next_idx=True)

    def group(gg, _):
        for b in range(GROUP):
            i = 1 + gg * GROUP + b
            step(i, 1 + b, first=False, next_gather=True, next_idx=True)
        return ()

    lax.fori_loop(0, (BLOCKS_PER_W - 6) // GROUP, group, ())

    for i in range(1 + ((BLOCKS_PER_W - 6) // GROUP) * GROUP, BLOCKS_PER_W):
        step(i, i, first=False,
             next_gather=(i + 1 < BLOCKS_PER_W),
             next_idx=(i + 3 < BLOCKS_PER_W))
    wait_write()


@jax.jit
def _run(ev_t, table):
    mesh = plsc.VectorSubcoreMesh(core_axis_name="c", subcore_axis_name="s")
    kern = functools.partial(
        pl.kernel,
        out_type=jax.ShapeDtypeStruct((NT, 8, NB // 128, 8, 128), jnp.float32),
        mesh=mesh,
        scratch_types=[
            pltpu.VMEM((NIDX, BBLK), jnp.int32),
            pltpu.VMEM((NROW, BBLK, 2 * EMBED), jnp.float32),
            pltpu.VMEM((BBLK // 128, EMBED, 129), jnp.float32),
            pltpu.SemaphoreType.DMA((NIDX,)),
            pltpu.SemaphoreType.DMA((NROW,)),
            pltpu.SemaphoreType.DMA,
        ],
        compiler_params=pltpu.CompilerParams(use_tc_tiling_on_sc=False,
                                             needs_layout_passes=False),
    )(_gather_body)
    return kern(ev_t, table)


def kernel(events, table):
    table_p = jnp.pad(table, ((0, 0), (0, 64)))
    out_p = _run(events.T, table_p)
    return out_p.transpose(2, 4, 0, 1, 3).reshape(NB, NT, EMBED)


# speedup vs baseline: 1.1108x; 1.1108x over previous
import functools

import jax
import jax.numpy as jnp
from jax import lax
from jax.experimental import pallas as pl
from jax.experimental.pallas import tpu as pltpu
from jax.experimental.pallas import tpu_sc as plsc

EMBED = 64
VOCAB_ROWS = 1000000
NB = 4096
NT = 200
NUM_WORKERS = 32
BBLK = 512
BLOCKS_PER_T = NB // BBLK
N_BLOCKS = NT * BLOCKS_PER_T
BLOCKS_PER_W = N_BLOCKS // NUM_WORKERS
NIDX = 4
NROW = 2
BBLK_P = BBLK + 1
GROUP = 4


def _gather_body(ev_hbm, table_hbm, out_hbm, idxv, rows, rows_t, isem, gsem,
                 wsem):
    wid = lax.axis_index("s") * 2 + lax.axis_index("c")
    k0 = wid * BLOCKS_PER_W

    iota16 = lax.iota(jnp.int32, 16)
    evecs = [q * 16 + iota16 for q in range(4)]

    def t_of(i):
        return (k0 + i) // BLOCKS_PER_T

    def b0_of(i):
        return ((k0 + i) % BLOCKS_PER_T) * BBLK

    def start_idx(i, s):
        pltpu.make_async_copy(
            ev_hbm.at[t_of(i), pl.ds(b0_of(i), BBLK)], idxv.at[s], isem.at[s]
        ).start()

    def wait_idx(s):
        pltpu.make_async_copy(
            ev_hbm.at[0, pl.ds(0, BBLK)], idxv.at[s], isem.at[s]
        ).wait()

    def start_gather(s, b):
        pltpu.make_async_copy(
            table_hbm.at[idxv.at[s]], rows.at[b], gsem.at[b]
        ).start()

    def wait_gather(b):
        pltpu.make_async_copy(
            table_hbm.at[idxv.at[0]], rows.at[b], gsem.at[b]
        ).wait()

    def start_write(i):
        t = t_of(i)
        c0 = b0_of(i) // 128

        def one(e, _):
            pltpu.make_async_copy(
                rows_t.at[:, pl.ds(e, 1), pl.ds(0, 128)],
                out_hbm.at[t, e // 8, pl.ds(c0, BBLK // 128),
                           pl.ds(e % 8, 1), :],
                wsem
            ).start()
            return ()

        lax.fori_loop(0, EMBED, one, ())

    def wait_write():
        def one(e, _):
            pltpu.make_async_copy(
                rows_t.at[:, pl.ds(e, 1), pl.ds(0, 128)],
                out_hbm.at[0, 0, pl.ds(0, BBLK // 128), pl.ds(0, 1), :], wsem
            ).wait()
            return ()

        lax.fori_loop(0, EMBED, one, ())

    def transpose(b):
        rows_b = rows.at[b]

        @plsc.parallel_loop(0, BBLK, step=1, unroll=4)
        def _(j):
            cv = jnp.full((16,), j // 128, jnp.int32)
            bv = jnp.full((16,), j % 128, jnp.int32)
            for q in range(4):
                val = rows_b[j, pl.ds(q * 16, 16)]
                plsc.store_scatter(rows_t, [cv, evecs[q], bv], val)

    def step(i, ib, *, first, next_gather, next_idx):
        if next_gather:
            wait_idx((ib + 1) % NIDX)
            start_gather((ib + 1) % NIDX, (ib + 1) % NROW)
        if next_idx:
            start_idx(i + 3, (ib + 3) % NIDX)
        wait_gather(ib % NROW)
        if not first:
            wait_write()
        transpose(ib % NROW)
        start_write(i)

    for s in range(3):
        start_idx(s, s)
    wait_idx(0)
    start_gather(0, 0)

    step(0, 0, first=True, next_gather=True, next_idx=True)

    def group(gg, _):
        for b in range(GROUP):
            i = 1 + gg * GROUP + b
            step(i, 1 + b, first=False, next_gather=True, next_idx=True)
        return ()

    lax.fori_loop(0, (BLOCKS_PER_W - 6) // GROUP, group, ())

    for i in range(1 + ((BLOCKS_PER_W - 6) // GROUP) * GROUP, BLOCKS_PER_W):
        step(i, i, first=False,
             next_gather=(i + 1 < BLOCKS_PER_W),
             next_idx=(i + 3 < BLOCKS_PER_W))
    wait_write()


@jax.jit
def _run(ev_t, table):
    mesh = plsc.VectorSubcoreMesh(core_axis_name="c", subcore_axis_name="s")
    kern = functools.partial(
        pl.kernel,
        out_type=jax.ShapeDtypeStruct((NT, 8, NB // 128, 8, 128), jnp.float32),
        mesh=mesh,
        scratch_types=[
            pltpu.VMEM((NIDX, BBLK), jnp.int32),
            pltpu.VMEM((NROW, BBLK, EMBED), jnp.float32),
            pltpu.VMEM((BBLK // 128, EMBED, 129), jnp.float32),
            pltpu.SemaphoreType.DMA((NIDX,)),
            pltpu.SemaphoreType.DMA((NROW,)),
            pltpu.SemaphoreType.DMA,
        ],
        compiler_params=pltpu.CompilerParams(use_tc_tiling_on_sc=False,
                                             needs_layout_passes=False),
    )(_gather_body)
    return kern(ev_t, table)


def kernel(events, table):
    table_p = jnp.pad(table, ((0, 0), (0, 64))).reshape(2 * VOCAB_ROWS, 64)
    out_p = _run(events.T * 2, table_p)
    return out_p.transpose(2, 4, 0, 1, 3).reshape(NB, NT, EMBED)
